# Initial kernel scaffold; baseline (speedup 1.0000x reference)
import jax
import jax.numpy as jnp
from jax.experimental import pallas as pl


def kernel(x_paper, x_author, edge_index_cites, edge_index_writes, edge_index_rev_writes, params):
    raise NotImplementedError("write your pallas kernel here")


# speedup vs baseline: 2.1569x; 2.1569x over previous
import functools

import jax
import jax.numpy as jnp
from jax import lax
from jax.experimental import pallas as pl
from jax.experimental.pallas import tpu as pltpu
from jax.experimental.pallas import tpu_sc as plsc

N = 50000
NPAD = 50048
E = 600000
EPAD = 602112
STRIPE = NPAD // 16
EPC = EPAD // 16
NCHUNK = EPC // 128
EPC2 = EPAD // 32
NCHUNK2 = EPC2 // 128
DUMMY_DST = N + 40
HID = 128
OUT_CH = 349
OUT_PAD = 384

@functools.lru_cache(maxsize=None)
def _mesh():
    return plsc.VectorSubcoreMesh(core_axis_name="c", subcore_axis_name="s")


def _agg_body(x_hbm, srcs_hbm, dsts_hbm, zeros_hbm, out_hbm,
              idx_v, dst_v, rows_v, acc_sh, sem):
    c = lax.axis_index("c")
    s = lax.axis_index("s")
    stripe0 = s * STRIPE
    sub_base = s * EPC
    for f in range(2):
        row_off = (c * 2 + f) * NPAD
        pltpu.sync_copy(zeros_hbm, acc_sh.at[pl.ds(stripe0, STRIPE)])
        plsc.subcore_barrier()

        def chunk(j, carry):
            eb = sub_base + j * 128
            pltpu.sync_copy(srcs_hbm.at[pl.ds(eb, 128)], idx_v)
            pltpu.sync_copy(dsts_hbm.at[pl.ds(eb, 128)], dst_v)
            for i in range(8):
                sl = pl.ds(i * 16, 16)
                idx_v[sl] = idx_v[sl] + row_off
            pltpu.async_copy(x_hbm.at[idx_v], rows_v, sem).wait()
            pltpu.sync_copy(rows_v, acc_sh.at[dst_v], add=True)
            return carry

        lax.fori_loop(0, NCHUNK, chunk, 0)
        plsc.subcore_barrier()
        pltpu.sync_copy(acc_sh.at[pl.ds(stripe0, STRIPE)],
                        out_hbm.at[pl.ds(row_off + stripe0, STRIPE)])
        plsc.subcore_barrier()


@functools.lru_cache(maxsize=None)
def _agg_kernel():
    return pl.kernel(
        _agg_body,
        out_type=jax.ShapeDtypeStruct((4 * NPAD, 32), jnp.float32),
        mesh=_mesh(),
        compiler_params=pltpu.CompilerParams(use_tc_tiling_on_sc=False),
        scratch_types=[
            pltpu.VMEM((128,), jnp.int32),
            pltpu.VMEM((128,), jnp.int32),
            pltpu.VMEM((128, 32), jnp.float32),
            pltpu.VMEM_SHARED((NPAD, 32), jnp.float32),
            pltpu.SemaphoreType.DMA,
        ],
    )


def _cnt_body(dsts_hbm, ones_hbm, zeros_hbm, out_hbm,
              dst_v, ones_v, acc_sh):
    c = lax.axis_index("c")
    s = lax.axis_index("s")
    stripe0 = s * STRIPE
    sub_base = c * (EPAD // 2) + s * EPC2
    pltpu.sync_copy(ones_hbm, ones_v)
    pltpu.sync_copy(zeros_hbm, acc_sh.at[pl.ds(stripe0, STRIPE)])
    plsc.subcore_barrier()

    def chunk(j, carry):
        eb = sub_base + j * 128
        pltpu.sync_copy(dsts_hbm.at[pl.ds(eb, 128)], dst_v)
        pltpu.sync_copy(ones_v, acc_sh.at[dst_v], add=True)
        return carry

    lax.fori_loop(0, NCHUNK2, chunk, 0)
    plsc.subcore_barrier()
    pltpu.sync_copy(acc_sh.at[pl.ds(stripe0, STRIPE)],
                    out_hbm.at[pl.ds(c * NPAD + stripe0, STRIPE)])


@functools.lru_cache(maxsize=None)
def _cnt_kernel():
    return pl.kernel(
        _cnt_body,
        out_type=jax.ShapeDtypeStruct((2 * NPAD, 16), jnp.float32),
        mesh=_mesh(),
        compiler_params=pltpu.CompilerParams(use_tc_tiling_on_sc=False),
        scratch_types=[
            pltpu.VMEM((128,), jnp.int32),
            pltpu.VMEM((128, 16), jnp.float32),
            pltpu.VMEM_SHARED((NPAD, 16), jnp.float32),
        ],
    )


_BLK = 3128


def _combine(aggs, cnts, x, wls, wr, b, wo=None, bo=None):
    n = len(aggs)
    final = wo is not None
    d_out = OUT_PAD if final else HID

    def body(*refs):
        k = 0
        agg_r = refs[k:k + n]; k += n
        cnt_r = refs[k:k + n]; k += n
        x_r = refs[k]; k += 1
        wl_r = refs[k:k + n]; k += n
        wr_r = refs[k]; k += 1
        b_r = refs[k]; k += 1
        if final:
            wo_r = refs[k]; k += 1
            bo_r = refs[k]; k += 1
        out_r = refs[k]
        acc = jnp.dot(x_r[...], wr_r[...], preferred_element_type=jnp.float32)
        for i in range(n):
            inv = 1.0 / jnp.maximum(cnt_r[i][...], 1.0)
            mean = agg_r[i][...] * inv
            acc = acc + jnp.dot(mean, wl_r[i][...],
                                preferred_element_type=jnp.float32)
        h = jnp.maximum(acc + b_r[...], 0.0)
        if final:
            out_r[...] = jnp.dot(h, wo_r[...],
                                 preferred_element_type=jnp.float32) + bo_r[...]
        else:
            out_r[...] = h

    in_specs = (
        [pl.BlockSpec((_BLK, HID), lambda i: (i, 0)) for _ in range(n)]
        + [pl.BlockSpec((_BLK, 1), lambda i: (i, 0)) for _ in range(n)]
        + [pl.BlockSpec((_BLK, HID), lambda i: (i, 0))]
        + [pl.BlockSpec((HID, HID), lambda i: (0, 0)) for _ in range(n + 1)]
        + [pl.BlockSpec((1, HID), lambda i: (0, 0))]
    )
    args = list(aggs) + list(cnts) + [x] + list(wls) + [wr, b]
    if final:
        in_specs += [pl.BlockSpec((HID, OUT_PAD), lambda i: (0, 0)),
                     pl.BlockSpec((1, OUT_PAD), lambda i: (0, 0))]
        args += [wo, bo]
    return pl.pallas_call(
        body,
        grid=(NPAD // _BLK,),
        in_specs=in_specs,
        out_specs=pl.BlockSpec((_BLK, d_out), lambda i: (i, 0)),
        out_shape=jax.ShapeDtypeStruct((NPAD, d_out), jnp.float32),
    )(*args)


def _slice_feats(x_pad):
    return x_pad.reshape(NPAD, 4, 32).transpose(1, 0, 2).reshape(4 * NPAD, 32)


def _unslice_feats(a):
    return a.reshape(4, NPAD, 32).transpose(1, 0, 2).reshape(NPAD, 128)


def _prep_edges(ei):
    src = ei[0].astype(jnp.int32)
    dst = ei[1].astype(jnp.int32)
    pad = EPAD - E
    src = jnp.concatenate([src, jnp.full((pad,), N, jnp.int32)])
    dst = jnp.concatenate([dst, jnp.full((pad,), DUMMY_DST, jnp.int32)])
    return src, dst


@jax.jit
def _forward(x_paper, x_author, ei_c, ei_w, ei_r, params):
    zeros32 = jnp.zeros((STRIPE, 32), jnp.float32)
    zeros16 = jnp.zeros((STRIPE, 16), jnp.float32)
    ones16 = jnp.ones((128, 16), jnp.float32)

    src_c, dst_c = _prep_edges(ei_c)
    src_w, dst_w = _prep_edges(ei_w)
    src_r, dst_r = _prep_edges(ei_r)

    xp = jnp.pad(x_paper, ((0, NPAD - N), (0, 0)))
    xa = jnp.pad(x_author, ((0, NPAD - N), (0, 0)))
    xp_sl = _slice_feats(xp)
    xa_sl = _slice_feats(xa)

    def counts(dst):
        part = _cnt_kernel()(dst, ones16, zeros16)
        return (part[:NPAD, :1] + part[NPAD:, :1])

    cnt_c = counts(dst_c)
    cnt_w = counts(dst_w)
    cnt_r = counts(dst_r)

    agg = _agg_kernel()
    agg_c = _unslice_feats(agg(xp_sl, src_c, dst_c, zeros32))
    agg_w = _unslice_feats(agg(xa_sl, src_w, dst_w, zeros32))
    agg_r = _unslice_feats(agg(xp_sl, src_r, dst_r, zeros32))

    p1 = _combine(
        [agg_c, agg_w], [cnt_c, cnt_w], xp,
        [params['l1_cites_Wl'], params['l1_writes_Wl']],
        params['l1_cites_Wr'] + params['l1_writes_Wr'],
        (params['l1_cites_bl'] + params['l1_writes_bl']).reshape(1, HID))
    a1 = _combine(
        [agg_r], [cnt_r], xa,
        [params['l1_rev_writes_Wl']],
        params['l1_rev_writes_Wr'],
        params['l1_rev_writes_bl'].reshape(1, HID))

    p1_sl = _slice_feats(p1)
    a1_sl = _slice_feats(a1)
    agg2_c = _unslice_feats(agg(p1_sl, src_c, dst_c, zeros32))
    agg2_w = _unslice_feats(agg(a1_sl, src_w, dst_w, zeros32))
    agg2_r = _unslice_feats(agg(p1_sl, src_r, dst_r, zeros32))

    wo_p = jnp.pad(params['lin_paper_W'], ((0, 0), (0, OUT_PAD - OUT_CH)))
    bo_p = jnp.pad(params['lin_paper_b'], (0, OUT_PAD - OUT_CH)).reshape(1, -1)
    wo_a = jnp.pad(params['lin_author_W'], ((0, 0), (0, OUT_PAD - OUT_CH)))
    bo_a = jnp.pad(params['lin_author_b'], (0, OUT_PAD - OUT_CH)).reshape(1, -1)

    out_p = _combine(
        [agg2_c, agg2_w], [cnt_c, cnt_w], p1,
        [params['l2_cites_Wl'], params['l2_writes_Wl']],
        params['l2_cites_Wr'] + params['l2_writes_Wr'],
        (params['l2_cites_bl'] + params['l2_writes_bl']).reshape(1, HID),
        wo=wo_p, bo=bo_p)
    out_a = _combine(
        [agg2_r], [cnt_r], a1,
        [params['l2_rev_writes_Wl']],
        params['l2_rev_writes_Wr'],
        params['l2_rev_writes_bl'].reshape(1, HID),
        wo=wo_a, bo=bo_a)

    return out_p[:N, :OUT_CH], out_a[:N, :OUT_CH]


def kernel(x_paper, x_author, edge_index_cites, edge_index_writes,
           edge_index_rev_writes, params):
    return _forward(x_paper, x_author, edge_index_cites, edge_index_writes,
                    edge_index_rev_writes, params)


# speedup vs baseline: 4.4140x; 2.0465x over previous
import functools

import jax
import jax.numpy as jnp
from jax import lax
from jax.experimental import pallas as pl
from jax.experimental.pallas import tpu as pltpu
from jax.experimental.pallas import tpu_sc as plsc

N = 50000
NPAD = 50048
E = 600000
EPAD = 602112
STRIPE = NPAD // 16
TOTCH = EPAD // 128
NCHUNK = TOTCH // 16
GRP = 3
NGRP = NCHUNK // GRP
NCHUNK2 = TOTCH // 32
DUMMY_DST = N + 40
HID = 128
OUT_CH = 349
OUT_PAD = 384

@functools.lru_cache(maxsize=None)
def _mesh():
    return plsc.VectorSubcoreMesh(core_axis_name="c", subcore_axis_name="s")


def _agg_body(x_hbm, srcs_hbm, dsts_hbm, zeros_hbm, out_hbm,
              idx_v, dst_v, rows_v, acc_sh, sem):
    c = lax.axis_index("c")
    s = lax.axis_index("s")
    stripe0 = s * STRIPE
    sub_ch0 = s * NCHUNK

    def fire(g, bbase, slice_off):
        row0 = sub_ch0 + g * GRP
        pltpu.sync_copy(srcs_hbm.at[pl.ds(slice_off + row0, GRP)],
                        idx_v.at[pl.ds(bbase, GRP)])
        pltpu.sync_copy(dsts_hbm.at[pl.ds(row0, GRP)],
                        dst_v.at[pl.ds(bbase, GRP)])
        for j in range(GRP):
            pltpu.async_copy(x_hbm.at[idx_v.at[bbase + j]],
                             rows_v.at[pl.ds((bbase + j) * 128, 128)], sem)

    for f in range(2):
        row_off = (c * 2 + f) * NPAD
        slice_off = (c * 2 + f) * TOTCH
        pltpu.sync_copy(zeros_hbm, acc_sh.at[pl.ds(stripe0, STRIPE)])
        plsc.subcore_barrier()

        fire(0, 0, slice_off)

        def group(i, carry):
            bcur = (i % 2) * GRP
            bnxt = ((i + 1) % 2) * GRP

            @pl.when(i + 1 < NGRP)
            def _():
                fire(i + 1, bnxt, slice_off)

            for j in range(GRP):
                pltpu.make_async_copy(
                    x_hbm.at[pl.ds(0, 128)],
                    rows_v.at[pl.ds((bcur + j) * 128, 128)], sem).wait()
            for j in range(GRP):
                pltpu.sync_copy(rows_v.at[pl.ds((bcur + j) * 128, 128)],
                                acc_sh.at[dst_v.at[bcur + j]], add=True)
            return carry

        lax.fori_loop(0, NGRP, group, 0)
        plsc.subcore_barrier()
        pltpu.sync_copy(acc_sh.at[pl.ds(stripe0, STRIPE)],
                        out_hbm.at[pl.ds(row_off + stripe0, STRIPE)])
        plsc.subcore_barrier()


@functools.lru_cache(maxsize=None)
def _agg_kernel():
    return pl.kernel(
        _agg_body,
        out_type=jax.ShapeDtypeStruct((4 * NPAD, 32), jnp.float32),
        mesh=_mesh(),
        compiler_params=pltpu.CompilerParams(use_tc_tiling_on_sc=False),
        scratch_types=[
            pltpu.VMEM((2 * GRP, 128), jnp.int32),
            pltpu.VMEM((2 * GRP, 128), jnp.int32),
            pltpu.VMEM((2 * GRP * 128, 32), jnp.float32),
            pltpu.VMEM_SHARED((NPAD, 32), jnp.float32),
            pltpu.SemaphoreType.DMA,
        ],
    )


def _cnt_body(dsts_hbm, ones_hbm, zeros_hbm, out_hbm,
              dst_v, ones_v, acc_sh):
    c = lax.axis_index("c")
    s = lax.axis_index("s")
    stripe0 = s * STRIPE
    sub_ch0 = c * (TOTCH // 2) + s * NCHUNK2
    pltpu.sync_copy(ones_hbm, ones_v)
    pltpu.sync_copy(zeros_hbm, acc_sh.at[pl.ds(stripe0, STRIPE)])
    plsc.subcore_barrier()

    def chunk(j, carry):
        pltpu.sync_copy(dsts_hbm.at[pl.ds(sub_ch0 + j, 1)], dst_v)
        pltpu.sync_copy(ones_v, acc_sh.at[dst_v.at[0]], add=True)
        return carry

    lax.fori_loop(0, NCHUNK2, chunk, 0)
    plsc.subcore_barrier()
    pltpu.sync_copy(acc_sh.at[pl.ds(stripe0, STRIPE)],
                    out_hbm.at[pl.ds(c * NPAD + stripe0, STRIPE)])


@functools.lru_cache(maxsize=None)
def _cnt_kernel():
    return pl.kernel(
        _cnt_body,
        out_type=jax.ShapeDtypeStruct((2 * NPAD, 16), jnp.float32),
        mesh=_mesh(),
        compiler_params=pltpu.CompilerParams(use_tc_tiling_on_sc=False),
        scratch_types=[
            pltpu.VMEM((1, 128), jnp.int32),
            pltpu.VMEM((128, 16), jnp.float32),
            pltpu.VMEM_SHARED((NPAD, 16), jnp.float32),
        ],
    )


_BLK = 3128


def _combine(aggs, cnts, x, wls, wr, b, wo=None, bo=None):
    n = len(aggs)
    final = wo is not None
    d_out = OUT_PAD if final else HID

    def body(*refs):
        k = 0
        agg_r = refs[k:k + n]; k += n
        cnt_r = refs[k:k + n]; k += n
        x_r = refs[k]; k += 1
        wl_r = refs[k:k + n]; k += n
        wr_r = refs[k]; k += 1
        b_r = refs[k]; k += 1
        if final:
            wo_r = refs[k]; k += 1
            bo_r = refs[k]; k += 1
        out_r = refs[k]
        acc = jnp.dot(x_r[...], wr_r[...], preferred_element_type=jnp.float32)
        for i in range(n):
            inv = 1.0 / jnp.maximum(cnt_r[i][...], 1.0)
            mean = agg_r[i][...] * inv
            acc = acc + jnp.dot(mean, wl_r[i][...],
                                preferred_element_type=jnp.float32)
        h = jnp.maximum(acc + b_r[...], 0.0)
        if final:
            out_r[...] = jnp.dot(h, wo_r[...],
                                 preferred_element_type=jnp.float32) + bo_r[...]
        else:
            out_r[...] = h

    in_specs = (
        [pl.BlockSpec((_BLK, HID), lambda i: (i, 0)) for _ in range(n)]
        + [pl.BlockSpec((_BLK, 1), lambda i: (i, 0)) for _ in range(n)]
        + [pl.BlockSpec((_BLK, HID), lambda i: (i, 0))]
        + [pl.BlockSpec((HID, HID), lambda i: (0, 0)) for _ in range(n + 1)]
        + [pl.BlockSpec((1, HID), lambda i: (0, 0))]
    )
    args = list(aggs) + list(cnts) + [x] + list(wls) + [wr, b]
    if final:
        in_specs += [pl.BlockSpec((HID, OUT_PAD), lambda i: (0, 0)),
                     pl.BlockSpec((1, OUT_PAD), lambda i: (0, 0))]
        args += [wo, bo]
    return pl.pallas_call(
        body,
        grid=(NPAD // _BLK,),
        in_specs=in_specs,
        out_specs=pl.BlockSpec((_BLK, d_out), lambda i: (i, 0)),
        out_shape=jax.ShapeDtypeStruct((NPAD, d_out), jnp.float32),
    )(*args)


def _slice_feats(x_pad):
    return x_pad.reshape(NPAD, 4, 32).transpose(1, 0, 2).reshape(4 * NPAD, 32)


def _unslice_feats(a):
    return a.reshape(4, NPAD, 32).transpose(1, 0, 2).reshape(NPAD, 128)


def _prep_edges(ei):
    src = ei[0].astype(jnp.int32)
    dst = ei[1].astype(jnp.int32)
    pad = EPAD - E
    src = jnp.concatenate([src, jnp.full((pad,), N, jnp.int32)])
    dst = jnp.concatenate([dst, jnp.full((pad,), DUMMY_DST, jnp.int32)])
    src_g = (src.reshape(1, TOTCH, 128)
             + (jnp.arange(4, dtype=jnp.int32) * NPAD).reshape(4, 1, 1))
    return src_g.reshape(4 * TOTCH, 128), dst.reshape(TOTCH, 128)


@jax.jit
def _forward(x_paper, x_author, ei_c, ei_w, ei_r, params):
    zeros32 = jnp.zeros((STRIPE, 32), jnp.float32)
    zeros16 = jnp.zeros((STRIPE, 16), jnp.float32)
    ones16 = jnp.ones((128, 16), jnp.float32)

    src_c, dst_c = _prep_edges(ei_c)
    src_w, dst_w = _prep_edges(ei_w)
    src_r, dst_r = _prep_edges(ei_r)

    xp = jnp.pad(x_paper, ((0, NPAD - N), (0, 0)))
    xa = jnp.pad(x_author, ((0, NPAD - N), (0, 0)))
    xp_sl = _slice_feats(xp)
    xa_sl = _slice_feats(xa)

    def counts(dst):
        part = _cnt_kernel()(dst, ones16, zeros16)
        return (part[:NPAD, :1] + part[NPAD:, :1])

    cnt_c = counts(dst_c)
    cnt_w = counts(dst_w)
    cnt_r = counts(dst_r)

    agg = _agg_kernel()
    agg_c = _unslice_feats(agg(xp_sl, src_c, dst_c, zeros32))
    agg_w = _unslice_feats(agg(xa_sl, src_w, dst_w, zeros32))
    agg_r = _unslice_feats(agg(xp_sl, src_r, dst_r, zeros32))

    p1 = _combine(
        [agg_c, agg_w], [cnt_c, cnt_w], xp,
        [params['l1_cites_Wl'], params['l1_writes_Wl']],
        params['l1_cites_Wr'] + params['l1_writes_Wr'],
        (params['l1_cites_bl'] + params['l1_writes_bl']).reshape(1, HID))
    a1 = _combine(
        [agg_r], [cnt_r], xa,
        [params['l1_rev_writes_Wl']],
        params['l1_rev_writes_Wr'],
        params['l1_rev_writes_bl'].reshape(1, HID))

    p1_sl = _slice_feats(p1)
    a1_sl = _slice_feats(a1)
    agg2_c = _unslice_feats(agg(p1_sl, src_c, dst_c, zeros32))
    agg2_w = _unslice_feats(agg(a1_sl, src_w, dst_w, zeros32))
    agg2_r = _unslice_feats(agg(p1_sl, src_r, dst_r, zeros32))

    wo_p = jnp.pad(params['lin_paper_W'], ((0, 0), (0, OUT_PAD - OUT_CH)))
    bo_p = jnp.pad(params['lin_paper_b'], (0, OUT_PAD - OUT_CH)).reshape(1, -1)
    wo_a = jnp.pad(params['lin_author_W'], ((0, 0), (0, OUT_PAD - OUT_CH)))
    bo_a = jnp.pad(params['lin_author_b'], (0, OUT_PAD - OUT_CH)).reshape(1, -1)

    out_p = _combine(
        [agg2_c, agg2_w], [cnt_c, cnt_w], p1,
        [params['l2_cites_Wl'], params['l2_writes_Wl']],
        params['l2_cites_Wr'] + params['l2_writes_Wr'],
        (params['l2_cites_bl'] + params['l2_writes_bl']).reshape(1, HID),
        wo=wo_p, bo=bo_p)
    out_a = _combine(
        [agg2_r], [cnt_r], a1,
        [params['l2_rev_writes_Wl']],
        params['l2_rev_writes_Wr'],
        params['l2_rev_writes_bl'].reshape(1, HID),
        wo=wo_a, bo=bo_a)

    return out_p[:N, :OUT_CH], out_a[:N, :OUT_CH]


def kernel(x_paper, x_author, edge_index_cites, edge_index_writes,
           edge_index_rev_writes, params):
    return _forward(x_paper, x_author, edge_index_cites, edge_index_writes,
                    edge_index_rev_writes, params)


# speedup vs baseline: 4.7023x; 1.0653x over previous
import functools

import jax
import jax.numpy as jnp
from jax import lax
from jax.experimental import pallas as pl
from jax.experimental.pallas import tpu as pltpu
from jax.experimental.pallas import tpu_sc as plsc

N = 50000
NPAD = 50048
E = 600000
EPAD = 602112
STRIPE = NPAD // 16
TOTCH = EPAD // 128
NCHUNK = TOTCH // 16
GRP = 3
NGRP = NCHUNK // GRP
NCHUNK2 = TOTCH // 32
GRP2 = 7
NGRP2 = NCHUNK2 // GRP2
DUMMY_DST = N + 40
HID = 128
OUT_CH = 349
OUT_PAD = 384

@functools.lru_cache(maxsize=None)
def _mesh():
    return plsc.VectorSubcoreMesh(core_axis_name="c", subcore_axis_name="s")


def _agg_body(x_hbm, srcs_hbm, dsts_hbm, zeros_hbm, out_hbm,
              idx_v, dst_v, rows_v, acc_sh, sem_g, sem_s):
    c = lax.axis_index("c")
    s = lax.axis_index("s")
    stripe0 = s * STRIPE
    sub_ch0 = s * NCHUNK

    def fire_gath(g, bbase, slice_off):
        row0 = sub_ch0 + g * GRP
        pltpu.sync_copy(srcs_hbm.at[pl.ds(slice_off + row0, GRP)],
                        idx_v.at[pl.ds(bbase, GRP)])
        pltpu.sync_copy(dsts_hbm.at[pl.ds(row0, GRP)],
                        dst_v.at[pl.ds(bbase, GRP)])
        for j in range(GRP):
            pltpu.async_copy(x_hbm.at[idx_v.at[bbase + j]],
                             rows_v.at[pl.ds((bbase + j) * 128, 128)], sem_g)

    def drain(sem, n):
        for _ in range(n):
            pltpu.make_async_copy(x_hbm.at[pl.ds(0, 128)],
                                  rows_v.at[pl.ds(0, 128)], sem).wait()

    for f in range(2):
        row_off = (c * 2 + f) * NPAD
        slice_off = (c * 2 + f) * TOTCH
        pltpu.sync_copy(zeros_hbm, acc_sh.at[pl.ds(stripe0, STRIPE)])
        plsc.subcore_barrier()

        fire_gath(0, 0, slice_off)

        def group(i, carry):
            bcur = (i % 2) * GRP
            bnxt = ((i + 1) % 2) * GRP

            @pl.when(i > 0)
            def _():
                drain(sem_s, GRP)

            @pl.when(i + 1 < NGRP)
            def _():
                fire_gath(i + 1, bnxt, slice_off)

            drain(sem_g, GRP)
            for j in range(GRP):
                pltpu.async_copy(rows_v.at[pl.ds((bcur + j) * 128, 128)],
                                 acc_sh.at[dst_v.at[bcur + j]], sem_s,
                                 add=True)
            return carry

        lax.fori_loop(0, NGRP, group, 0)
        drain(sem_s, GRP)
        plsc.subcore_barrier()
        pltpu.sync_copy(acc_sh.at[pl.ds(stripe0, STRIPE)],
                        out_hbm.at[pl.ds(row_off + stripe0, STRIPE)])
        plsc.subcore_barrier()


@functools.lru_cache(maxsize=None)
def _agg_kernel():
    return pl.kernel(
        _agg_body,
        out_type=jax.ShapeDtypeStruct((4 * NPAD, 32), jnp.float32),
        mesh=_mesh(),
        compiler_params=pltpu.CompilerParams(use_tc_tiling_on_sc=False),
        scratch_types=[
            pltpu.VMEM((2 * GRP, 128), jnp.int32),
            pltpu.VMEM((2 * GRP, 128), jnp.int32),
            pltpu.VMEM((2 * GRP * 128, 32), jnp.float32),
            pltpu.VMEM_SHARED((NPAD, 32), jnp.float32),
            pltpu.SemaphoreType.DMA,
            pltpu.SemaphoreType.DMA,
        ],
    )


def _cnt_body(dsts_hbm, ones_hbm, zeros_hbm, out_hbm,
              dst_v, ones_v, acc_sh, sem_s):
    c = lax.axis_index("c")
    s = lax.axis_index("s")
    stripe0 = s * STRIPE
    sub_ch0 = c * (TOTCH // 2) + s * NCHUNK2
    pltpu.sync_copy(ones_hbm, ones_v)
    pltpu.sync_copy(zeros_hbm, acc_sh.at[pl.ds(stripe0, STRIPE)])
    plsc.subcore_barrier()

    def load(g, bbase):
        pltpu.sync_copy(dsts_hbm.at[pl.ds(sub_ch0 + g * GRP2, GRP2)],
                        dst_v.at[pl.ds(bbase, GRP2)])

    def drain(n):
        for _ in range(n):
            pltpu.make_async_copy(ones_hbm, ones_v, sem_s).wait()

    load(0, 0)

    def group(i, carry):
        bcur = (i % 2) * GRP2
        bnxt = ((i + 1) % 2) * GRP2

        @pl.when(i > 0)
        def _():
            drain(GRP2)

        @pl.when(i + 1 < NGRP2)
        def _():
            load(i + 1, bnxt)

        for j in range(GRP2):
            pltpu.async_copy(ones_v, acc_sh.at[dst_v.at[bcur + j]], sem_s,
                             add=True)
        return carry

    lax.fori_loop(0, NGRP2, group, 0)
    drain(GRP2)
    plsc.subcore_barrier()
    pltpu.sync_copy(acc_sh.at[pl.ds(stripe0, STRIPE)],
                    out_hbm.at[pl.ds(c * NPAD + stripe0, STRIPE)])


@functools.lru_cache(maxsize=None)
def _cnt_kernel():
    return pl.kernel(
        _cnt_body,
        out_type=jax.ShapeDtypeStruct((2 * NPAD, 16), jnp.float32),
        mesh=_mesh(),
        compiler_params=pltpu.CompilerParams(use_tc_tiling_on_sc=False),
        scratch_types=[
            pltpu.VMEM((2 * GRP2, 128), jnp.int32),
            pltpu.VMEM((128, 16), jnp.float32),
            pltpu.VMEM_SHARED((NPAD, 16), jnp.float32),
            pltpu.SemaphoreType.DMA,
        ],
    )


_BLK = 3128


def _combine(aggs, cnts, x, wls, wr, b, wo=None, bo=None):
    n = len(aggs)
    final = wo is not None
    d_out = OUT_PAD if final else HID

    def body(*refs):
        k = 0
        agg_r = refs[k:k + n]; k += n
        cnt_r = refs[k:k + n]; k += n
        x_r = refs[k]; k += 1
        wl_r = refs[k:k + n]; k += n
        wr_r = refs[k]; k += 1
        b_r = refs[k]; k += 1
        if final:
            wo_r = refs[k]; k += 1
            bo_r = refs[k]; k += 1
        out_r = refs[k]
        acc = jnp.dot(x_r[...], wr_r[...], preferred_element_type=jnp.float32)
        for i in range(n):
            inv = 1.0 / jnp.maximum(cnt_r[i][...], 1.0)
            mean = agg_r[i][...] * inv
            acc = acc + jnp.dot(mean, wl_r[i][...],
                                preferred_element_type=jnp.float32)
        h = jnp.maximum(acc + b_r[...], 0.0)
        if final:
            out_r[...] = jnp.dot(h, wo_r[...],
                                 preferred_element_type=jnp.float32) + bo_r[...]
        else:
            out_r[...] = h

    in_specs = (
        [pl.BlockSpec((_BLK, HID), lambda i: (i, 0)) for _ in range(n)]
        + [pl.BlockSpec((_BLK, 1), lambda i: (i, 0)) for _ in range(n)]
        + [pl.BlockSpec((_BLK, HID), lambda i: (i, 0))]
        + [pl.BlockSpec((HID, HID), lambda i: (0, 0)) for _ in range(n + 1)]
        + [pl.BlockSpec((1, HID), lambda i: (0, 0))]
    )
    args = list(aggs) + list(cnts) + [x] + list(wls) + [wr, b]
    if final:
        in_specs += [pl.BlockSpec((HID, OUT_PAD), lambda i: (0, 0)),
                     pl.BlockSpec((1, OUT_PAD), lambda i: (0, 0))]
        args += [wo, bo]
    return pl.pallas_call(
        body,
        grid=(NPAD // _BLK,),
        in_specs=in_specs,
        out_specs=pl.BlockSpec((_BLK, d_out), lambda i: (i, 0)),
        out_shape=jax.ShapeDtypeStruct((NPAD, d_out), jnp.float32),
    )(*args)


def _slice_feats(x_pad):
    return x_pad.reshape(NPAD, 4, 32).transpose(1, 0, 2).reshape(4 * NPAD, 32)


def _unslice_feats(a):
    return a.reshape(4, NPAD, 32).transpose(1, 0, 2).reshape(NPAD, 128)


def _prep_edges(ei):
    src = ei[0].astype(jnp.int32)
    dst = ei[1].astype(jnp.int32)
    pad = EPAD - E
    src = jnp.concatenate([src, jnp.full((pad,), N, jnp.int32)])
    dst = jnp.concatenate([dst, jnp.full((pad,), DUMMY_DST, jnp.int32)])
    src_g = (src.reshape(1, TOTCH, 128)
             + (jnp.arange(4, dtype=jnp.int32) * NPAD).reshape(4, 1, 1))
    return src_g.reshape(4 * TOTCH, 128), dst.reshape(TOTCH, 128)


@jax.jit
def _forward(x_paper, x_author, ei_c, ei_w, ei_r, params):
    zeros32 = jnp.zeros((STRIPE, 32), jnp.float32)
    zeros16 = jnp.zeros((STRIPE, 16), jnp.float32)
    ones16 = jnp.ones((128, 16), jnp.float32)

    src_c, dst_c = _prep_edges(ei_c)
    src_w, dst_w = _prep_edges(ei_w)
    src_r, dst_r = _prep_edges(ei_r)

    xp = jnp.pad(x_paper, ((0, NPAD - N), (0, 0)))
    xa = jnp.pad(x_author, ((0, NPAD - N), (0, 0)))
    xp_sl = _slice_feats(xp)
    xa_sl = _slice_feats(xa)

    def counts(dst):
        part = _cnt_kernel()(dst, ones16, zeros16)
        return (part[:NPAD, :1] + part[NPAD:, :1])

    cnt_c = counts(dst_c)
    cnt_w = counts(dst_w)
    cnt_r = counts(dst_r)

    agg = _agg_kernel()
    agg_c = _unslice_feats(agg(xp_sl, src_c, dst_c, zeros32))
    agg_w = _unslice_feats(agg(xa_sl, src_w, dst_w, zeros32))
    agg_r = _unslice_feats(agg(xp_sl, src_r, dst_r, zeros32))

    p1 = _combine(
        [agg_c, agg_w], [cnt_c, cnt_w], xp,
        [params['l1_cites_Wl'], params['l1_writes_Wl']],
        params['l1_cites_Wr'] + params['l1_writes_Wr'],
        (params['l1_cites_bl'] + params['l1_writes_bl']).reshape(1, HID))
    a1 = _combine(
        [agg_r], [cnt_r], xa,
        [params['l1_rev_writes_Wl']],
        params['l1_rev_writes_Wr'],
        params['l1_rev_writes_bl'].reshape(1, HID))

    p1_sl = _slice_feats(p1)
    a1_sl = _slice_feats(a1)
    agg2_c = _unslice_feats(agg(p1_sl, src_c, dst_c, zeros32))
    agg2_w = _unslice_feats(agg(a1_sl, src_w, dst_w, zeros32))
    agg2_r = _unslice_feats(agg(p1_sl, src_r, dst_r, zeros32))

    wo_p = jnp.pad(params['lin_paper_W'], ((0, 0), (0, OUT_PAD - OUT_CH)))
    bo_p = jnp.pad(params['lin_paper_b'], (0, OUT_PAD - OUT_CH)).reshape(1, -1)
    wo_a = jnp.pad(params['lin_author_W'], ((0, 0), (0, OUT_PAD - OUT_CH)))
    bo_a = jnp.pad(params['lin_author_b'], (0, OUT_PAD - OUT_CH)).reshape(1, -1)

    out_p = _combine(
        [agg2_c, agg2_w], [cnt_c, cnt_w], p1,
        [params['l2_cites_Wl'], params['l2_writes_Wl']],
        params['l2_cites_Wr'] + params['l2_writes_Wr'],
        (params['l2_cites_bl'] + params['l2_writes_bl']).reshape(1, HID),
        wo=wo_p, bo=bo_p)
    out_a = _combine(
        [agg2_r], [cnt_r], a1,
        [params['l2_rev_writes_Wl']],
        params['l2_rev_writes_Wr'],
        params['l2_rev_writes_bl'].reshape(1, HID),
        wo=wo_a, bo=bo_a)

    return out_p[:N, :OUT_CH], out_a[:N, :OUT_CH]


def kernel(x_paper, x_author, edge_index_cites, edge_index_writes,
           edge_index_rev_writes, params):
    return _forward(x_paper, x_author, edge_index_cites, edge_index_writes,
                    edge_index_rev_writes, params)


# speedup vs baseline: 4.8419x; 1.0297x over previous
import functools

import jax
import jax.numpy as jnp
from jax import lax
from jax.experimental import pallas as pl
from jax.experimental.pallas import tpu as pltpu
from jax.experimental.pallas import tpu_sc as plsc

N = 50000
NPAD = 50048
E = 600000
EPAD = 602112
STRIPE = NPAD // 16
TOTCH = EPAD // 128
NCHUNK = TOTCH // 16
GRP = 3
NGRP = NCHUNK // GRP
NCHUNK2 = TOTCH // 32
GRP2 = 7
NGRP2 = NCHUNK2 // GRP2
DUMMY_DST = N + 40
HID = 128
OUT_CH = 349
OUT_PAD = 384

@functools.lru_cache(maxsize=None)
def _mesh():
    return plsc.VectorSubcoreMesh(core_axis_name="c", subcore_axis_name="s")


def _agg_body(x_hbm, srcs_hbm, dsts_hbm, zeros_hbm, out_hbm,
              idx_v, dst_v, rows_v, acc_sh, sem_g, sem_s):
    c = lax.axis_index("c")
    s = lax.axis_index("s")
    stripe0 = s * STRIPE
    sub_ch0 = s * NCHUNK

    def fire_gath(g, bbase, slice_off):
        row0 = sub_ch0 + g * GRP
        pltpu.sync_copy(srcs_hbm.at[pl.ds(slice_off + row0, GRP)],
                        idx_v.at[pl.ds(bbase, GRP)])
        pltpu.sync_copy(dsts_hbm.at[pl.ds(row0, GRP)],
                        dst_v.at[pl.ds(bbase, GRP)])
        for j in range(GRP):
            pltpu.async_copy(x_hbm.at[idx_v.at[bbase + j]],
                             rows_v.at[pl.ds((bbase + j) * 128, 128)], sem_g)

    def drain(sem, n):
        for _ in range(n):
            pltpu.make_async_copy(x_hbm.at[pl.ds(0, 128)],
                                  rows_v.at[pl.ds(0, 128)], sem).wait()

    for f in range(2):
        row_off = (c * 2 + f) * NPAD
        slice_off = (c * 2 + f) * TOTCH
        pltpu.sync_copy(zeros_hbm, acc_sh.at[pl.ds(stripe0, STRIPE)])
        plsc.subcore_barrier()

        fire_gath(0, 0, slice_off)

        def group(i, carry):
            bcur = (i % 2) * GRP
            bnxt = ((i + 1) % 2) * GRP

            @pl.when(i > 0)
            def _():
                drain(sem_s, GRP)

            @pl.when(i + 1 < NGRP)
            def _():
                fire_gath(i + 1, bnxt, slice_off)

            drain(sem_g, GRP)
            for j in range(GRP):
                pltpu.async_copy(rows_v.at[pl.ds((bcur + j) * 128, 128)],
                                 acc_sh.at[dst_v.at[bcur + j]], sem_s,
                                 add=True)
            return carry

        lax.fori_loop(0, NGRP, group, 0)
        drain(sem_s, GRP)
        plsc.subcore_barrier()
        pltpu.sync_copy(acc_sh.at[pl.ds(stripe0, STRIPE)],
                        out_hbm.at[pl.ds(row_off + stripe0, STRIPE)])
        plsc.subcore_barrier()


@functools.lru_cache(maxsize=None)
def _agg_kernel():
    return pl.kernel(
        _agg_body,
        out_type=jax.ShapeDtypeStruct((4 * NPAD, 32), jnp.float32),
        mesh=_mesh(),
        compiler_params=pltpu.CompilerParams(use_tc_tiling_on_sc=False),
        scratch_types=[
            pltpu.VMEM((2 * GRP, 128), jnp.int32),
            pltpu.VMEM((2 * GRP, 128), jnp.int32),
            pltpu.VMEM((2 * GRP * 128, 32), jnp.float32),
            pltpu.VMEM_SHARED((NPAD, 32), jnp.float32),
            pltpu.SemaphoreType.DMA,
            pltpu.SemaphoreType.DMA,
        ],
    )


def _cnt_body(dsts_hbm, ones_hbm, zeros_hbm, out_hbm,
              dst_v, ones_v, acc_sh, sem_s):
    c = lax.axis_index("c")
    s = lax.axis_index("s")
    stripe0 = s * STRIPE
    sub_ch0 = c * (TOTCH // 2) + s * NCHUNK2
    pltpu.sync_copy(ones_hbm, ones_v)
    pltpu.sync_copy(zeros_hbm, acc_sh.at[pl.ds(stripe0, STRIPE)])
    plsc.subcore_barrier()

    def load(g, bbase):
        pltpu.sync_copy(dsts_hbm.at[pl.ds(sub_ch0 + g * GRP2, GRP2)],
                        dst_v.at[pl.ds(bbase, GRP2)])

    def drain(n):
        for _ in range(n):
            pltpu.make_async_copy(ones_hbm, ones_v, sem_s).wait()

    load(0, 0)

    def group(i, carry):
        bcur = (i % 2) * GRP2
        bnxt = ((i + 1) % 2) * GRP2

        @pl.when(i > 0)
        def _():
            drain(GRP2)

        @pl.when(i + 1 < NGRP2)
        def _():
            load(i + 1, bnxt)

        for j in range(GRP2):
            pltpu.async_copy(ones_v, acc_sh.at[dst_v.at[bcur + j]], sem_s,
                             add=True)
        return carry

    lax.fori_loop(0, NGRP2, group, 0)
    drain(GRP2)
    plsc.subcore_barrier()
    pltpu.sync_copy(acc_sh.at[pl.ds(stripe0, STRIPE)],
                    out_hbm.at[pl.ds(c * NPAD + stripe0, STRIPE)])


@functools.lru_cache(maxsize=None)
def _cnt_kernel():
    return pl.kernel(
        _cnt_body,
        out_type=jax.ShapeDtypeStruct((2 * NPAD, 16), jnp.float32),
        mesh=_mesh(),
        compiler_params=pltpu.CompilerParams(use_tc_tiling_on_sc=False),
        scratch_types=[
            pltpu.VMEM((2 * GRP2, 128), jnp.int32),
            pltpu.VMEM((128, 16), jnp.float32),
            pltpu.VMEM_SHARED((NPAD, 16), jnp.float32),
            pltpu.SemaphoreType.DMA,
        ],
    )


_BLK = 3128


def _xproj(x, w, b, d_out):
    def body(x_r, w_r, b_r, out_r):
        out_r[...] = jnp.dot(x_r[...], w_r[...],
                             preferred_element_type=jnp.float32) + b_r[...]

    return pl.pallas_call(
        body,
        grid=(NPAD // _BLK,),
        in_specs=[pl.BlockSpec((_BLK, HID), lambda i: (i, 0)),
                  pl.BlockSpec((HID, d_out), lambda i: (0, 0)),
                  pl.BlockSpec((1, d_out), lambda i: (0, 0))],
        out_specs=pl.BlockSpec((_BLK, d_out), lambda i: (i, 0)),
        out_shape=jax.ShapeDtypeStruct((NPAD, d_out), jnp.float32),
    )(x, w, b)


def _combine(aggs, cnts, xr, wls, wo=None, bo=None):
    n = len(aggs)
    final = wo is not None
    d_out = OUT_PAD if final else HID

    def body(*refs):
        k = 0
        agg_r = refs[k:k + 4 * n]; k += 4 * n
        cnt_r = refs[k:k + n]; k += n
        xr_r = refs[k]; k += 1
        wl_r = refs[k:k + n]; k += n
        if final:
            wo_r = refs[k]; k += 1
            bo_r = refs[k]; k += 1
        out_r = refs[k]
        acc = xr_r[...]
        for i in range(n):
            inv = 1.0 / jnp.maximum(cnt_r[i][...], 1.0)
            mean = jnp.concatenate([agg_r[4 * i + j][...] for j in range(4)],
                                   axis=1) * inv
            acc = acc + jnp.dot(mean, wl_r[i][...],
                                preferred_element_type=jnp.float32)
        h = jnp.maximum(acc, 0.0)
        if final:
            out_r[...] = jnp.dot(h, wo_r[...],
                                 preferred_element_type=jnp.float32) + bo_r[...]
        else:
            out_r[...] = h

    in_specs = (
        [pl.BlockSpec((_BLK, 32), lambda i: (i, 0)) for _ in range(4 * n)]
        + [pl.BlockSpec((_BLK, 1), lambda i: (i, 0)) for _ in range(n)]
        + [pl.BlockSpec((_BLK, HID), lambda i: (i, 0))]
        + [pl.BlockSpec((HID, HID), lambda i: (0, 0)) for _ in range(n)]
    )
    args = ([a[j * NPAD:(j + 1) * NPAD] for a in aggs for j in range(4)]
            + list(cnts) + [xr] + list(wls))
    if final:
        in_specs += [pl.BlockSpec((HID, OUT_PAD), lambda i: (0, 0)),
                     pl.BlockSpec((1, OUT_PAD), lambda i: (0, 0))]
        args += [wo, bo]
    return pl.pallas_call(
        body,
        grid=(NPAD // _BLK,),
        in_specs=in_specs,
        out_specs=pl.BlockSpec((_BLK, d_out), lambda i: (i, 0)),
        out_shape=jax.ShapeDtypeStruct((NPAD, d_out), jnp.float32),
    )(*args)


def _slice_feats(x_pad):
    return x_pad.reshape(NPAD, 4, 32).transpose(1, 0, 2).reshape(4 * NPAD, 32)


def _prep_edges(ei):
    src = ei[0].astype(jnp.int32)
    dst = ei[1].astype(jnp.int32)
    pad = EPAD - E
    src = jnp.concatenate([src, jnp.full((pad,), N, jnp.int32)])
    dst = jnp.concatenate([dst, jnp.full((pad,), DUMMY_DST, jnp.int32)])
    src_g = (src.reshape(1, TOTCH, 128)
             + (jnp.arange(4, dtype=jnp.int32) * NPAD).reshape(4, 1, 1))
    return src_g.reshape(4 * TOTCH, 128), dst.reshape(TOTCH, 128)


@jax.jit
def _forward(x_paper, x_author, ei_c, ei_w, ei_r, params):
    zeros32 = jnp.zeros((STRIPE, 32), jnp.float32)
    zeros16 = jnp.zeros((STRIPE, 16), jnp.float32)
    ones16 = jnp.ones((128, 16), jnp.float32)

    src_c, dst_c = _prep_edges(ei_c)
    src_w, dst_w = _prep_edges(ei_w)
    src_r, dst_r = _prep_edges(ei_r)

    xp = jnp.pad(x_paper, ((0, NPAD - N), (0, 0)))
    xa = jnp.pad(x_author, ((0, NPAD - N), (0, 0)))
    xp_sl = _slice_feats(xp)
    xa_sl = _slice_feats(xa)

    def counts(dst):
        part = _cnt_kernel()(dst, ones16, zeros16)
        return (part[:NPAD, :1] + part[NPAD:, :1])

    cnt_c = counts(dst_c)
    cnt_w = counts(dst_w)
    cnt_r = counts(dst_r)

    xr1_p = _xproj(xp, params['l1_cites_Wr'] + params['l1_writes_Wr'],
                   (params['l1_cites_bl'] + params['l1_writes_bl'])
                   .reshape(1, HID), HID)
    xr1_a = _xproj(xa, params['l1_rev_writes_Wr'],
                   params['l1_rev_writes_bl'].reshape(1, HID), HID)

    agg = _agg_kernel()
    agg_c = agg(xp_sl, src_c, dst_c, zeros32)
    agg_w = agg(xa_sl, src_w, dst_w, zeros32)
    agg_r = agg(xp_sl, src_r, dst_r, zeros32)

    p1 = _combine([agg_c, agg_w], [cnt_c, cnt_w], xr1_p,
                  [params['l1_cites_Wl'], params['l1_writes_Wl']])
    a1 = _combine([agg_r], [cnt_r], xr1_a,
                  [params['l1_rev_writes_Wl']])

    xr2_p = _xproj(p1, params['l2_cites_Wr'] + params['l2_writes_Wr'],
                   (params['l2_cites_bl'] + params['l2_writes_bl'])
                   .reshape(1, HID), HID)
    xr2_a = _xproj(a1, params['l2_rev_writes_Wr'],
                   params['l2_rev_writes_bl'].reshape(1, HID), HID)

    p1_sl = _slice_feats(p1)
    a1_sl = _slice_feats(a1)
    agg2_c = agg(p1_sl, src_c, dst_c, zeros32)
    agg2_w = agg(a1_sl, src_w, dst_w, zeros32)
    agg2_r = agg(p1_sl, src_r, dst_r, zeros32)

    wo_p = jnp.pad(params['lin_paper_W'], ((0, 0), (0, OUT_PAD - OUT_CH)))
    bo_p = jnp.pad(params['lin_paper_b'], (0, OUT_PAD - OUT_CH)).reshape(1, -1)
    wo_a = jnp.pad(params['lin_author_W'], ((0, 0), (0, OUT_PAD - OUT_CH)))
    bo_a = jnp.pad(params['lin_author_b'], (0, OUT_PAD - OUT_CH)).reshape(1, -1)

    out_p = _combine([agg2_c, agg2_w], [cnt_c, cnt_w], xr2_p,
                     [params['l2_cites_Wl'], params['l2_writes_Wl']],
                     wo=wo_p, bo=bo_p)
    out_a = _combine([agg2_r], [cnt_r], xr2_a,
                     [params['l2_rev_writes_Wl']],
                     wo=wo_a, bo=bo_a)

    return out_p[:N, :OUT_CH], out_a[:N, :OUT_CH]


def kernel(x_paper, x_author, edge_index_cites, edge_index_writes,
           edge_index_rev_writes, params):
    return _forward(x_paper, x_author, edge_index_cites, edge_index_writes,
                    edge_index_rev_writes, params)
